# baseline (speedup 1.0000x reference)
import functools
import math

import jax
import jax.numpy as jnp
from jax import lax
from jax.experimental import pallas as pl
from jax.experimental.pallas import tpu as pltpu
from jax.experimental.pallas import tpu_sc as plsc

_VOCAB = 100000
_EMBED = 64
_CONTEXT = 2
_BATCH = 1024

_NUM_WORKERS = 32
_ROWS_PER_WORKER = _BATCH // _NUM_WORKERS

_GATHER_W = 128
_BB = 8


def _sc_gather(table_hbm, idx_hbm, out_hbm, idx_v, rows_v, sem):
    wid = lax.axis_index("s") * 2 + lax.axis_index("c")
    base = wid * _ROWS_PER_WORKER
    pltpu.sync_copy(idx_hbm.at[pl.ds(base, _ROWS_PER_WORKER)], idx_v)
    pltpu.async_copy(table_hbm.at[idx_v], rows_v, sem).wait()
    pltpu.sync_copy(rows_v, out_hbm.at[pl.ds(base, _ROWS_PER_WORKER)])


def _tc_body(embx_ref, w_hbm, b_ref, o_ref, w_vmem, sem):
    i = pl.program_id(0)

    @pl.when(i == 0)
    def _load_w():
        cp = pltpu.make_async_copy(w_hbm, w_vmem, sem)
        cp.start()
        cp.wait()

    z = lax.dot_general(
        embx_ref[pl.ds(i * _BB, _BB), :], w_vmem[...],
        dimension_numbers=(((1,), (1,)), ((), ())),
        preferred_element_type=jnp.float32,
    ) + b_ref[...]
    shifted = z - z
    log_prob = shifted - jnp.log(jnp.exp(shifted) + jnp.exp(shifted))
    o_ref[:, 0, :] = log_prob
    o_ref[:, 1, :] = log_prob


def kernel(x, emb_table, W, b):
    mesh = plsc.VectorSubcoreMesh(core_axis_name="c", subcore_axis_name="s")
    gather = functools.partial(
        pl.kernel,
        mesh=mesh,
        out_type=jax.ShapeDtypeStruct((_BATCH, _GATHER_W), jnp.float32),
        scratch_types=[
            pltpu.VMEM((_ROWS_PER_WORKER,), jnp.int32),
            pltpu.VMEM((_ROWS_PER_WORKER, _GATHER_W), jnp.float32),
            pltpu.SemaphoreType.DMA,
        ],
    )(_sc_gather)
    table_pad = jnp.pad(emb_table, ((0, 0), (0, _GATHER_W - _EMBED)))
    embx = gather(table_pad, x)

    b2d = b.reshape(1, _VOCAB)
    embx64 = embx[:, :_EMBED]
    return pl.pallas_call(
        _tc_body,
        grid=(_BATCH // _BB,),
        in_specs=[
            pl.BlockSpec((_BATCH, _EMBED), lambda i: (0, 0)),
            pl.BlockSpec(memory_space=pl.ANY),
            pl.BlockSpec((1, _VOCAB), lambda i: (0, 0)),
        ],
        out_specs=pl.BlockSpec((_BB, _CONTEXT, _VOCAB), lambda i: (i, 0, 0)),
        out_shape=jax.ShapeDtypeStruct((_BATCH, _CONTEXT, _VOCAB), jnp.float32),
        scratch_shapes=[
            pltpu.VMEM((_VOCAB, _EMBED), jnp.float32),
            pltpu.SemaphoreType.DMA,
        ],
        compiler_params=pltpu.CompilerParams(vmem_limit_bytes=120 * 1024 * 1024),
    )(embx64, W, b2d)


# speedup vs baseline: 1.5030x; 1.5030x over previous
import math

import jax
import jax.numpy as jnp
from jax.experimental import pallas as pl

_VOCAB = 100000
_CONTEXT = 2
_BATCH_BLOCK = 8


def _fill_body(o_ref):
    o_ref[...] = jnp.full(o_ref.shape, -math.log(2.0), dtype=jnp.float32)


def kernel(x, emb_table, W, b):
    batch = x.shape[0]
    return pl.pallas_call(
        _fill_body,
        grid=(batch // _BATCH_BLOCK,),
        out_specs=pl.BlockSpec((_BATCH_BLOCK, _CONTEXT, _VOCAB), lambda i: (i, 0, 0)),
        out_shape=jax.ShapeDtypeStruct((batch, _CONTEXT, _VOCAB), jnp.float32),
    )()
